# baseline (speedup 1.0000x reference)
import functools

import jax
import jax.numpy as jnp
from jax import lax
from jax.experimental import pallas as pl
from jax.experimental.pallas import tpu as pltpu
from jax.experimental.pallas import tpu_sc as plsc

N = 10000
E = 320000
D = 128
H = 128
C = 40
L = 5
DT = 0.05

NP = 10240
NTILES = 16
ROWS_PT = NP // NTILES
CHUNK = 128
CPT = 160
TOT_CHUNKS = CPT * NTILES
EP = TOT_CHUNKS * CHUNK
IBUF = 32
NBLK = CPT // IBUF

_mesh = plsc.VectorSubcoreMesh(core_axis_name="c", subcore_axis_name="s")


@functools.partial(
    pl.kernel,
    mesh=_mesh,
    out_type=jax.ShapeDtypeStruct((NP, H), jnp.float32),
    scratch_types=[
        pltpu.VMEM((IBUF, CHUNK), jnp.int32),
        pltpu.VMEM((CHUNK, H), jnp.float32),
        pltpu.VMEM_SHARED((NP, H), jnp.float32),
    ],
)
def _deg_kernel(dst_hbm, z_hbm, ones_hbm, deg_out, dst_idx, ones_v, deg_sh):
    cid = lax.axis_index("c")
    sid = lax.axis_index("s")
    row0 = sid * ROWS_PT
    pltpu.sync_copy(z_hbm.at[pl.ds(row0, ROWS_PT)], deg_sh.at[pl.ds(row0, ROWS_PT)])
    pltpu.sync_copy(ones_hbm, ones_v)
    plsc.subcore_barrier()

    @pl.when(cid == 0)
    def _():
        @pl.loop(0, NBLK)
        def _(b):
            c0 = sid * CPT + b * IBUF
            pltpu.sync_copy(dst_hbm.at[pl.ds(c0, IBUF)], dst_idx)

            @pl.loop(0, IBUF)
            def _(j):
                pltpu.sync_copy(ones_v, deg_sh.at[dst_idx.at[j]], add=True)

    plsc.subcore_barrier()

    @pl.when(cid == 0)
    def _():
        pltpu.sync_copy(deg_sh.at[pl.ds(row0, ROWS_PT)], deg_out.at[pl.ds(row0, ROWS_PT)])


@functools.partial(
    pl.kernel,
    mesh=_mesh,
    out_type=(
        jax.ShapeDtypeStruct((NP, H), jnp.float32),
        jax.ShapeDtypeStruct((NP, H), jnp.float32),
    ),
    scratch_types=[
        pltpu.VMEM((IBUF, CHUNK), jnp.int32),
        pltpu.VMEM((IBUF, CHUNK), jnp.int32),
        pltpu.VMEM((CHUNK, H), jnp.float32),
        pltpu.VMEM((CHUNK, H), jnp.float32),
        pltpu.VMEM_SHARED((NP, H), jnp.float32),
        pltpu.SemaphoreType.DMA,
        pltpu.SemaphoreType.DMA,
    ],
)
def _agg_kernel(x_hbm, y_hbm, src_hbm, dst_hbm, z_hbm,
                aggx_out, aggy_out, src_idx, dst_idx, rows0, rows1, agg_sh,
                sem0, sem1):
    cid = lax.axis_index("c")
    sid = lax.axis_index("s")
    row0 = sid * ROWS_PT
    pltpu.sync_copy(z_hbm.at[pl.ds(row0, ROWS_PT)], agg_sh.at[pl.ds(row0, ROWS_PT)])
    plsc.subcore_barrier()

    def run(f_hbm):
        @pl.loop(0, NBLK)
        def _(b):
            c0 = sid * CPT + b * IBUF
            pltpu.sync_copy(src_hbm.at[pl.ds(c0, IBUF)], src_idx)
            pltpu.sync_copy(dst_hbm.at[pl.ds(c0, IBUF)], dst_idx)

            @pl.loop(0, IBUF, step=2)
            def _(j):
                cp0 = pltpu.async_copy(f_hbm.at[src_idx.at[j]], rows0, sem0)
                cp1 = pltpu.async_copy(f_hbm.at[src_idx.at[j + 1]], rows1, sem1)
                cp0.wait()
                pltpu.sync_copy(rows0, agg_sh.at[dst_idx.at[j]], add=True)
                cp1.wait()
                pltpu.sync_copy(rows1, agg_sh.at[dst_idx.at[j + 1]], add=True)

    @pl.when(cid == 0)
    def _():
        run(x_hbm)

    @pl.when(cid == 1)
    def _():
        run(y_hbm)

    plsc.subcore_barrier()

    @pl.when(cid == 0)
    def _():
        pltpu.sync_copy(agg_sh.at[pl.ds(row0, ROWS_PT)], aggx_out.at[pl.ds(row0, ROWS_PT)])

    @pl.when(cid == 1)
    def _():
        pltpu.sync_copy(agg_sh.at[pl.ds(row0, ROWS_PT)], aggy_out.at[pl.ds(row0, ROWS_PT)])


_BLK = 512
_GRID = NP // _BLK


def _lift_body(x_ref, wx_ref, bx_ref, wy_ref, by_ref, xo_ref, yo_ref):
    xb = x_ref[...]
    xo_ref[...] = jnp.tanh(
        jnp.dot(xb, wx_ref[...], preferred_element_type=jnp.float32) + bx_ref[...])
    yo_ref[...] = jnp.tanh(
        jnp.dot(xb, wy_ref[...], preferred_element_type=jnp.float32) + by_ref[...])


_lift = pl.pallas_call(
    _lift_body,
    grid=(_GRID,),
    in_specs=[
        pl.BlockSpec((_BLK, D), lambda i: (i, 0)),
        pl.BlockSpec((D, H), lambda i: (0, 0)),
        pl.BlockSpec((1, H), lambda i: (0, 0)),
        pl.BlockSpec((D, H), lambda i: (0, 0)),
        pl.BlockSpec((1, H), lambda i: (0, 0)),
    ],
    out_specs=(
        pl.BlockSpec((_BLK, H), lambda i: (i, 0)),
        pl.BlockSpec((_BLK, H), lambda i: (i, 0)),
    ),
    out_shape=(
        jax.ShapeDtypeStruct((NP, H), jnp.float32),
        jax.ShapeDtypeStruct((NP, H), jnp.float32),
    ),
)


def _update_body(x_ref, y_ref, ax_ref, ay_ref, deg_ref,
                 al_ref, be_ref, ga_ref, de_ref, xn_ref, yn_ref):
    invd = 1.0 / jnp.maximum(deg_ref[...], 1.0)
    xb = x_ref[...]
    yb = y_ref[...]
    max_ = ax_ref[...] * invd
    may_ = ay_ref[...] * invd
    xn_ref[...] = xb + DT * xb * (al_ref[...] - be_ref[...] * may_)
    yn_ref[...] = yb + DT * yb * (-ga_ref[...] + de_ref[...] * max_)


_update = pl.pallas_call(
    _update_body,
    grid=(_GRID,),
    in_specs=[
        pl.BlockSpec((_BLK, H), lambda i: (i, 0)),
        pl.BlockSpec((_BLK, H), lambda i: (i, 0)),
        pl.BlockSpec((_BLK, H), lambda i: (i, 0)),
        pl.BlockSpec((_BLK, H), lambda i: (i, 0)),
        pl.BlockSpec((_BLK, 1), lambda i: (i, 0)),
        pl.BlockSpec((1, H), lambda i: (0, 0)),
        pl.BlockSpec((1, H), lambda i: (0, 0)),
        pl.BlockSpec((1, H), lambda i: (0, 0)),
        pl.BlockSpec((1, H), lambda i: (0, 0)),
    ],
    out_specs=(
        pl.BlockSpec((_BLK, H), lambda i: (i, 0)),
        pl.BlockSpec((_BLK, H), lambda i: (i, 0)),
    ),
    out_shape=(
        jax.ShapeDtypeStruct((NP, H), jnp.float32),
        jax.ShapeDtypeStruct((NP, H), jnp.float32),
    ),
)


def _readout_body(x_ref, y_ref, wx_ref, wy_ref, br_ref, o_ref):
    o_ref[...] = (
        jnp.dot(x_ref[...], wx_ref[...], preferred_element_type=jnp.float32)
        + jnp.dot(y_ref[...], wy_ref[...], preferred_element_type=jnp.float32)
        + br_ref[...])


_readout = pl.pallas_call(
    _readout_body,
    grid=(_GRID,),
    in_specs=[
        pl.BlockSpec((_BLK, H), lambda i: (i, 0)),
        pl.BlockSpec((_BLK, H), lambda i: (i, 0)),
        pl.BlockSpec((H, 128), lambda i: (0, 0)),
        pl.BlockSpec((H, 128), lambda i: (0, 0)),
        pl.BlockSpec((1, 128), lambda i: (0, 0)),
    ],
    out_specs=pl.BlockSpec((_BLK, 128), lambda i: (i, 0)),
    out_shape=jax.ShapeDtypeStruct((NP, 128), jnp.float32),
)


def kernel(x, edge_index, Wx, bx, Wy, by, alpha, beta, gamma, delta, Wr, br):
    f32 = jnp.float32
    x_pad = jnp.pad(x, ((0, NP - N), (0, 0)))
    order = jnp.argsort(edge_index[0])
    src_s = edge_index[0][order]
    dst_s = edge_index[1][order]
    src2d = jnp.pad(src_s, (0, EP - E)).reshape(TOT_CHUNKS, CHUNK)
    dst2d = jnp.pad(dst_s, (0, EP - E), constant_values=N).reshape(TOT_CHUNKS, CHUNK)
    zeros128 = jnp.zeros((NP, H), f32)
    ones128 = jnp.ones((CHUNK, H), f32)

    X, Y = _lift(x_pad, Wx, bx.reshape(1, H), Wy, by.reshape(1, H))
    deg = _deg_kernel(dst2d, zeros128, ones128)
    deg1 = deg[:, :1]

    for l in range(L):
        aggX, aggY = _agg_kernel(X, Y, src2d, dst2d, zeros128)
        X, Y = _update(X, Y, aggX, aggY, deg1,
                       alpha[l].reshape(1, H), beta[l].reshape(1, H),
                       gamma[l].reshape(1, H), delta[l].reshape(1, H))

    WrX = jnp.pad(Wr[:H], ((0, 0), (0, 128 - C)))
    WrY = jnp.pad(Wr[H:], ((0, 0), (0, 128 - C)))
    brp = jnp.pad(br, (0, 128 - C)).reshape(1, 128)
    out = _readout(X, Y, WrX, WrY, brp)
    return out[:N, :C]


# speedup vs baseline: 1.2667x; 1.2667x over previous
import functools

import jax
import jax.numpy as jnp
from jax import lax
from jax.experimental import pallas as pl
from jax.experimental.pallas import tpu as pltpu
from jax.experimental.pallas import tpu_sc as plsc

N = 10000
E = 320000
D = 128
H = 128
C = 40
L = 5
DT = 0.05

NP = 10240
NTILES = 16
ROWS_PT = NP // NTILES
CHUNK = 128
CPT = 160
TOT_CHUNKS = CPT * NTILES
EP = TOT_CHUNKS * CHUNK
IBUF = 32
NBLK = CPT // IBUF

_mesh = plsc.VectorSubcoreMesh(core_axis_name="c", subcore_axis_name="s")


@functools.partial(
    pl.kernel,
    mesh=_mesh,
    out_type=jax.ShapeDtypeStruct((NP, H), jnp.float32),
    scratch_types=[
        pltpu.VMEM((IBUF, CHUNK), jnp.int32),
        pltpu.VMEM((CHUNK, H), jnp.float32),
        pltpu.VMEM_SHARED((NP, H), jnp.float32),
    ],
)
def _deg_kernel(dst_hbm, z_hbm, ones_hbm, deg_out, dst_idx, ones_v, deg_sh):
    cid = lax.axis_index("c")
    sid = lax.axis_index("s")
    row0 = sid * ROWS_PT
    pltpu.sync_copy(z_hbm.at[pl.ds(row0, ROWS_PT)], deg_sh.at[pl.ds(row0, ROWS_PT)])
    pltpu.sync_copy(ones_hbm, ones_v)
    plsc.subcore_barrier()

    @pl.when(cid == 0)
    def _():
        @pl.loop(0, NBLK)
        def _(b):
            c0 = sid * CPT + b * IBUF
            pltpu.sync_copy(dst_hbm.at[pl.ds(c0, IBUF)], dst_idx)

            @pl.loop(0, IBUF)
            def _(j):
                pltpu.sync_copy(ones_v, deg_sh.at[dst_idx.at[j]], add=True)

    plsc.subcore_barrier()

    @pl.when(cid == 0)
    def _():
        pltpu.sync_copy(deg_sh.at[pl.ds(row0, ROWS_PT)], deg_out.at[pl.ds(row0, ROWS_PT)])


@functools.partial(
    pl.kernel,
    mesh=_mesh,
    out_type=(
        jax.ShapeDtypeStruct((NP, H), jnp.float32),
        jax.ShapeDtypeStruct((NP, H), jnp.float32),
    ),
    scratch_types=[
        pltpu.VMEM((IBUF, CHUNK), jnp.int32),
        pltpu.VMEM((IBUF, CHUNK), jnp.int32),
        pltpu.VMEM((CHUNK, H), jnp.float32),
        pltpu.VMEM((CHUNK, H), jnp.float32),
        pltpu.VMEM_SHARED((NP, H), jnp.float32),
        pltpu.SemaphoreType.DMA,
    ],
)
def _agg_kernel(x_hbm, y_hbm, src_hbm, dst_hbm, z_hbm,
                aggx_out, aggy_out, src_idx, dst_idx, rows0, rows1, agg_sh,
                sem):
    cid = lax.axis_index("c")
    sid = lax.axis_index("s")
    row0 = sid * ROWS_PT
    pltpu.sync_copy(z_hbm.at[pl.ds(row0, ROWS_PT)], agg_sh.at[pl.ds(row0, ROWS_PT)])
    plsc.subcore_barrier()

    def run(f_hbm):
        @pl.loop(0, NBLK)
        def _(b):
            c0 = sid * CPT + b * IBUF
            pltpu.sync_copy(src_hbm.at[pl.ds(c0, IBUF)], src_idx)
            pltpu.sync_copy(dst_hbm.at[pl.ds(c0, IBUF)], dst_idx)

            @pl.loop(0, IBUF, step=2)
            def _(j):
                cp0 = pltpu.async_copy(f_hbm.at[src_idx.at[j]], rows0, sem)
                cp1 = pltpu.async_copy(f_hbm.at[src_idx.at[j + 1]], rows1, sem)
                cp0.wait()
                cp1.wait()
                pltpu.sync_copy(rows0, agg_sh.at[dst_idx.at[j]], add=True)
                pltpu.sync_copy(rows1, agg_sh.at[dst_idx.at[j + 1]], add=True)

    @pl.when(cid == 0)
    def _():
        run(x_hbm)

    @pl.when(cid == 1)
    def _():
        run(y_hbm)

    plsc.subcore_barrier()

    @pl.when(cid == 0)
    def _():
        pltpu.sync_copy(agg_sh.at[pl.ds(row0, ROWS_PT)], aggx_out.at[pl.ds(row0, ROWS_PT)])

    @pl.when(cid == 1)
    def _():
        pltpu.sync_copy(agg_sh.at[pl.ds(row0, ROWS_PT)], aggy_out.at[pl.ds(row0, ROWS_PT)])


_BLK = 512
_GRID = NP // _BLK


def _lift_body(x_ref, wx_ref, bx_ref, wy_ref, by_ref, xo_ref, yo_ref):
    xb = x_ref[...]
    xo_ref[...] = jnp.tanh(
        jnp.dot(xb, wx_ref[...], preferred_element_type=jnp.float32) + bx_ref[...])
    yo_ref[...] = jnp.tanh(
        jnp.dot(xb, wy_ref[...], preferred_element_type=jnp.float32) + by_ref[...])


_lift = pl.pallas_call(
    _lift_body,
    grid=(_GRID,),
    in_specs=[
        pl.BlockSpec((_BLK, D), lambda i: (i, 0)),
        pl.BlockSpec((D, H), lambda i: (0, 0)),
        pl.BlockSpec((1, H), lambda i: (0, 0)),
        pl.BlockSpec((D, H), lambda i: (0, 0)),
        pl.BlockSpec((1, H), lambda i: (0, 0)),
    ],
    out_specs=(
        pl.BlockSpec((_BLK, H), lambda i: (i, 0)),
        pl.BlockSpec((_BLK, H), lambda i: (i, 0)),
    ),
    out_shape=(
        jax.ShapeDtypeStruct((NP, H), jnp.float32),
        jax.ShapeDtypeStruct((NP, H), jnp.float32),
    ),
)


def _update_body(x_ref, y_ref, ax_ref, ay_ref, deg_ref,
                 al_ref, be_ref, ga_ref, de_ref, xn_ref, yn_ref):
    invd = 1.0 / jnp.maximum(deg_ref[...], 1.0)
    xb = x_ref[...]
    yb = y_ref[...]
    max_ = ax_ref[...] * invd
    may_ = ay_ref[...] * invd
    xn_ref[...] = xb + DT * xb * (al_ref[...] - be_ref[...] * may_)
    yn_ref[...] = yb + DT * yb * (-ga_ref[...] + de_ref[...] * max_)


_update = pl.pallas_call(
    _update_body,
    grid=(_GRID,),
    in_specs=[
        pl.BlockSpec((_BLK, H), lambda i: (i, 0)),
        pl.BlockSpec((_BLK, H), lambda i: (i, 0)),
        pl.BlockSpec((_BLK, H), lambda i: (i, 0)),
        pl.BlockSpec((_BLK, H), lambda i: (i, 0)),
        pl.BlockSpec((_BLK, 1), lambda i: (i, 0)),
        pl.BlockSpec((1, H), lambda i: (0, 0)),
        pl.BlockSpec((1, H), lambda i: (0, 0)),
        pl.BlockSpec((1, H), lambda i: (0, 0)),
        pl.BlockSpec((1, H), lambda i: (0, 0)),
    ],
    out_specs=(
        pl.BlockSpec((_BLK, H), lambda i: (i, 0)),
        pl.BlockSpec((_BLK, H), lambda i: (i, 0)),
    ),
    out_shape=(
        jax.ShapeDtypeStruct((NP, H), jnp.float32),
        jax.ShapeDtypeStruct((NP, H), jnp.float32),
    ),
)


def _readout_body(x_ref, y_ref, wx_ref, wy_ref, br_ref, o_ref):
    o_ref[...] = (
        jnp.dot(x_ref[...], wx_ref[...], preferred_element_type=jnp.float32)
        + jnp.dot(y_ref[...], wy_ref[...], preferred_element_type=jnp.float32)
        + br_ref[...])


_readout = pl.pallas_call(
    _readout_body,
    grid=(_GRID,),
    in_specs=[
        pl.BlockSpec((_BLK, H), lambda i: (i, 0)),
        pl.BlockSpec((_BLK, H), lambda i: (i, 0)),
        pl.BlockSpec((H, 128), lambda i: (0, 0)),
        pl.BlockSpec((H, 128), lambda i: (0, 0)),
        pl.BlockSpec((1, 128), lambda i: (0, 0)),
    ],
    out_specs=pl.BlockSpec((_BLK, 128), lambda i: (i, 0)),
    out_shape=jax.ShapeDtypeStruct((NP, 128), jnp.float32),
)


def kernel(x, edge_index, Wx, bx, Wy, by, alpha, beta, gamma, delta, Wr, br):
    f32 = jnp.float32
    x_pad = jnp.pad(x, ((0, NP - N), (0, 0)))
    src2d = jnp.pad(edge_index[0], (0, EP - E)).reshape(TOT_CHUNKS, CHUNK)
    dst2d = jnp.pad(edge_index[1], (0, EP - E), constant_values=N).reshape(TOT_CHUNKS, CHUNK)
    zeros128 = jnp.zeros((NP, H), f32)
    ones128 = jnp.ones((CHUNK, H), f32)

    X, Y = _lift(x_pad, Wx, bx.reshape(1, H), Wy, by.reshape(1, H))
    deg = _deg_kernel(dst2d, zeros128, ones128)
    deg1 = deg[:, :1]

    for l in range(L):
        aggX, aggY = _agg_kernel(X, Y, src2d, dst2d, zeros128)
        X, Y = _update(X, Y, aggX, aggY, deg1,
                       alpha[l].reshape(1, H), beta[l].reshape(1, H),
                       gamma[l].reshape(1, H), delta[l].reshape(1, H))

    WrX = jnp.pad(Wr[:H], ((0, 0), (0, 128 - C)))
    WrY = jnp.pad(Wr[H:], ((0, 0), (0, 128 - C)))
    brp = jnp.pad(br, (0, 128 - C)).reshape(1, 128)
    out = _readout(X, Y, WrX, WrY, brp)
    return out[:N, :C]
